# Initial kernel scaffold; baseline (speedup 1.0000x reference)
import functools

import jax
import jax.numpy as jnp
from jax import lax
from jax.experimental import pallas as pl
from jax.experimental.pallas import tpu as pltpu
from jax.experimental.pallas import tpu_sc as plsc

N_EMBD = 64
T = 200
B = 1024
NC = 2
NS = 16
NW = NC * NS
RPW = B // NW
LANES = 16
EV = N_EMBD // LANES


def _body(idx0_hbm, idx1_hbm, tab0_hbm, tab1_hbm, pos_hbm,
          out0_hbm, out1_hbm,
          pos_v, idx_v, buf, g0, g1, o0, o1):
    wid = lax.axis_index("s") * NC + lax.axis_index("c")
    base = wid * RPW

    pltpu.sync_copy(pos_hbm, pos_v)

    gsems = (g0, g1)
    osems = (o0, o1)
    pending_out = [None, None]

    def add_pos(p):
        @functools.partial(plsc.parallel_loop, 0, T, unroll=4)
        def _(i):
            for j in range(EV):
                sl = pl.ds(j * LANES, LANES)
                plsc.addupdate(buf.at[p, i, sl], pos_v[i, sl])

    for idx_hbm, tab_hbm, out_hbm in (
        (idx0_hbm, tab0_hbm, out0_hbm),
        (idx1_hbm, tab1_hbm, out1_hbm),
    ):
        pltpu.sync_copy(idx_hbm.at[pl.ds(base, RPW)], idx_v)

        def start_gather(r):
            p = r % 2
            if pending_out[p] is not None:
                pending_out[p].wait()
                pending_out[p] = None
            return pltpu.async_copy(tab_hbm.at[idx_v.at[r]], buf.at[p], gsems[p])

        g = start_gather(0)
        for r in range(RPW):
            p = r % 2
            g_next = start_gather(r + 1) if r + 1 < RPW else None
            g.wait()
            add_pos(p)
            pending_out[p] = pltpu.async_copy(
                buf.at[p], out_hbm.at[base + r], osems[p])
            g = g_next

    for p in range(2):
        if pending_out[p] is not None:
            pending_out[p].wait()
            pending_out[p] = None


@jax.jit
def _run(idx0, idx1, table0, table1, pos_table):
    mesh = plsc.VectorSubcoreMesh(core_axis_name="c", subcore_axis_name="s")
    f = pl.kernel(
        _body,
        out_type=[
            jax.ShapeDtypeStruct((B, T, N_EMBD), jnp.float32),
            jax.ShapeDtypeStruct((B, T, N_EMBD), jnp.float32),
        ],
        mesh=mesh,
        scratch_types=[
            pltpu.VMEM((T, N_EMBD), jnp.float32),
            pltpu.VMEM((RPW, T), jnp.int32),
            pltpu.VMEM((2, T, N_EMBD), jnp.float32),
            pltpu.SemaphoreType.DMA,
            pltpu.SemaphoreType.DMA,
            pltpu.SemaphoreType.DMA,
            pltpu.SemaphoreType.DMA,
        ],
    )
    return f(idx0, idx1, table0, table1, pos_table)


def kernel(idx0, idx1, table0, table1, pos_table):
    idx0 = idx0.astype(jnp.int32)
    idx1 = idx1.astype(jnp.int32)
    out0, out1 = _run(idx0, idx1, table0, table1, pos_table)
    return (out0, out1)


# speedup vs baseline: 1.1173x; 1.1173x over previous
import functools

import jax
import jax.numpy as jnp
from jax import lax
from jax.experimental import pallas as pl
from jax.experimental.pallas import tpu as pltpu
from jax.experimental.pallas import tpu_sc as plsc

N_EMBD = 64
T = 200
B = 1024
NC = 2
NS = 16
NW = NC * NS
CH = 128
CPW = (B * T) // (NW * CH)
LANES = 16
EV = N_EMBD // LANES


def _body(idx0_hbm, idx1_hbm, tab0_hbm, tab1_hbm, pos_hbm,
          out0_hbm, out1_hbm,
          pos2_v, idx_v, buf, g0, g1, o0, o1):
    wid = lax.axis_index("s") * NC + lax.axis_index("c")
    cbase = wid * CPW

    pltpu.sync_copy(pos_hbm, pos2_v.at[pl.ds(0, T)])
    pltpu.sync_copy(pos_hbm, pos2_v.at[pl.ds(T, T)])

    gsems = (g0, g1)
    osems = (o0, o1)

    def add_pos(b, t0):
        @plsc.parallel_loop(0, CH, unroll=8)
        def _(i):
            t = t0 + i
            for jj in range(EV):
                sl = pl.ds(jj * LANES, LANES)
                plsc.addupdate(buf.at[b, i, sl], pos2_v[t, sl])

    for idx_hbm, tab_hbm, out_hbm in (
        (idx0_hbm, tab0_hbm, out0_hbm),
        (idx1_hbm, tab1_hbm, out1_hbm),
    ):
        pltpu.sync_copy(idx_hbm.at[wid], idx_v)

        def gather(j, b):
            return pltpu.async_copy(tab_hbm.at[idx_v.at[j]], buf.at[b],
                                    gsems[b])

        def scatter(j, b):
            return pltpu.async_copy(
                buf.at[b], out_hbm.at[pl.ds((cbase + j) * CH, CH)], osems[b])

        def consume(j, b, t0):
            pltpu.make_async_copy(tab_hbm.at[idx_v.at[j]], buf.at[b],
                                  gsems[b]).wait()
            add_pos(b, t0)
            return scatter(j, b)

        gather(0, 0)
        gather(1, 1)

        @pl.loop(0, CPW - 2, step=2)
        def _(g):
            outs = []
            for b in range(2):
                j = g + b
                outs.append(consume(j, b, lax.rem(j * CH, T)))
            for b in range(2):
                outs[b].wait()
                gather(g + b + 2, b)

        tail = [consume(CPW - 2 + b, b, ((CPW - 2 + b) * CH) % T)
                for b in range(2)]
        for d in tail:
            d.wait()


@jax.jit
def _run(idx0f, idx1f, table0, table1, pos_table):
    mesh = plsc.VectorSubcoreMesh(core_axis_name="c", subcore_axis_name="s")
    f = pl.kernel(
        _body,
        out_type=[
            jax.ShapeDtypeStruct((B * T, N_EMBD), jnp.float32),
            jax.ShapeDtypeStruct((B * T, N_EMBD), jnp.float32),
        ],
        mesh=mesh,
        compiler_params=pltpu.CompilerParams(use_tc_tiling_on_sc=False),
        scratch_types=[
            pltpu.VMEM((2 * T, N_EMBD), jnp.float32),
            pltpu.VMEM((CPW, CH), jnp.int32),
            pltpu.VMEM((2, CH, N_EMBD), jnp.float32),
            pltpu.SemaphoreType.DMA,
            pltpu.SemaphoreType.DMA,
            pltpu.SemaphoreType.DMA,
            pltpu.SemaphoreType.DMA,
        ],
    )
    return f(idx0f, idx1f, table0, table1, pos_table)


def kernel(idx0, idx1, table0, table1, pos_table):
    idx0f = idx0.astype(jnp.int32).reshape(NW, CPW, CH)
    idx1f = idx1.astype(jnp.int32).reshape(NW, CPW, CH)
    out0, out1 = _run(idx0f, idx1f, table0, table1, pos_table)
    return (out0.reshape(B, T, N_EMBD), out1.reshape(B, T, N_EMBD))


# speedup vs baseline: 1.1343x; 1.0151x over previous
import jax
import jax.numpy as jnp
from jax import lax
from jax.experimental import pallas as pl
from jax.experimental.pallas import tpu as pltpu
from jax.experimental.pallas import tpu_sc as plsc

N_EMBD = 64
T = 200
B = 1024
NW = 32
LANES = 16
NBT = B // 128
NCH = T * NBT
CPW = NCH // NW
NTT = 9
TPAD = T + NTT


def _body(y0_hbm, y1_hbm, tab0_hbm, tab1_hbm, pos_splat_hbm,
          z0_hbm, z1_hbm,
          ibuf, gbuf, obuf, pos_v,
          g0, g1, o0, o1, i0, i1, i2, i3):
    wid = lax.axis_index("s") * 2 + lax.axis_index("c")
    cbase = wid * CPW
    t_first = (wid * CPW) // NBT

    gsems = (g0, g1)
    osems = (o0, o1)
    isems = (i0, i1, i2, i3)

    pltpu.sync_copy(
        pos_splat_hbm.at[pl.ds(t_first * N_EMBD * LANES, NTT * N_EMBD * LANES)],
        pos_v)

    iota = lax.iota(jnp.int32, LANES)

    for y_hbm, tab_hbm, z_hbm in (
        (y0_hbm, tab0_hbm, z0_hbm),
        (y1_hbm, tab1_hbm, z1_hbm),
    ):
        def coords(j):
            c = cbase + j
            return c // NBT, lax.rem(c, NBT)

        def fetch_idx(j, s):
            t, bt = coords(j)
            row = (t // 8) * 64 + bt * 8 + lax.rem(t, 8)
            pltpu.async_copy(y_hbm.at[pl.ds(row, 1)],
                             ibuf.at[pl.ds(s, 1)], isems[s])

        def start_gather(s, p):
            pltpu.make_async_copy(
                y_hbm.at[pl.ds(0, 1)], ibuf.at[pl.ds(0, 1)], isems[s]).wait()
            pltpu.async_copy(tab_hbm.at[ibuf.at[s]], gbuf.at[p], gsems[p])

        def consume(j, p):
            t, bt = coords(j)
            pltpu.make_async_copy(
                tab_hbm.at[ibuf.at[0]], gbuf.at[p], gsems[p]).wait()
            pbase = (t - t_first) * (N_EMBD * LANES)

            @plsc.parallel_loop(0, N_EMBD, unroll=4)
            def _(e):
                pv = pos_v[pl.ds(pbase + e * LANES, LANES)]
                cols = jnp.full((LANES,), e, jnp.int32)
                et = e // 8
                ei = lax.rem(e, 8)
                for bg in range(8):
                    rows = iota + (bg * LANES)
                    v = plsc.load_gather(gbuf.at[p], [rows, cols])
                    obuf[p, et, ei, pl.ds(bg * LANES, LANES)] = v + pv

            tile0 = (t * 8) * NBT + bt
            for et in range(8):
                pltpu.async_copy(
                    obuf.at[p, et],
                    z_hbm.at[pl.ds((tile0 + et * NBT) * 8, 8)], osems[p])

        def wait_out(p):
            for et in range(8):
                pltpu.make_async_copy(
                    obuf.at[p, et], z_hbm.at[pl.ds(0, 8)], osems[p]).wait()

        for j in range(4):
            fetch_idx(j, j)
        for j in range(2):
            start_gather(j, j)

        @pl.loop(0, CPW - 2, step=4)
        def _(g):
            for k in range(4):
                j = g + k
                p = k % 2

                @pl.when(j >= 2)
                def _():
                    wait_out(p)

                consume(j, p)

                @pl.when(j + 4 < CPW)
                def _():
                    fetch_idx(j + 4, k)

                start_gather((k + 2) % 4, p)

        for p in range(2):
            wait_out(p)
            consume(CPW - 2 + p, p)
        for p in range(2):
            wait_out(p)


@jax.jit
def _run(y0, y1, table0, table1, pos_splat):
    mesh = plsc.VectorSubcoreMesh(core_axis_name="c", subcore_axis_name="s")
    zshape = jax.ShapeDtypeStruct((T * 8 * NBT * 8, 128), jnp.float32)
    f = pl.kernel(
        _body,
        out_type=[zshape, zshape],
        mesh=mesh,
        compiler_params=pltpu.CompilerParams(
            use_tc_tiling_on_sc=False, needs_layout_passes=False),
        scratch_types=[
            pltpu.VMEM((4, 128), jnp.int32),
            pltpu.VMEM((2, 128, N_EMBD), jnp.float32),
            pltpu.VMEM((2, 8, 8, 128), jnp.float32),
            pltpu.VMEM((NTT * N_EMBD * LANES,), jnp.float32),
            pltpu.SemaphoreType.DMA,
            pltpu.SemaphoreType.DMA,
            pltpu.SemaphoreType.DMA,
            pltpu.SemaphoreType.DMA,
            pltpu.SemaphoreType.DMA,
            pltpu.SemaphoreType.DMA,
            pltpu.SemaphoreType.DMA,
            pltpu.SemaphoreType.DMA,
        ],
    )
    return f(y0, y1, table0, table1, pos_splat)


def kernel(idx0, idx1, table0, table1, pos_table):
    def as_tiles(idx):
        idx = idx.astype(jnp.int32)
        return (idx.T.reshape(T // 8, 8, NBT, 128)
                .transpose(0, 2, 1, 3).reshape(NCH, 128))

    pos_pad = jnp.pad(pos_table, ((0, NTT), (0, 0)))
    pos_splat = jnp.broadcast_to(
        pos_pad[:, :, None], (TPAD, N_EMBD, LANES)).reshape(-1)

    z0, z1 = _run(as_tiles(idx0), as_tiles(idx1), table0, table1, pos_splat)

    def as_out(z):
        return (z.reshape(T, 8, NBT, 8, 128)
                .transpose(2, 4, 0, 1, 3).reshape(B, T, N_EMBD))

    return (as_out(z0), as_out(z1))


# speedup vs baseline: 1.5017x; 1.3240x over previous
import jax
import jax.numpy as jnp
from jax import lax
from jax.experimental import pallas as pl
from jax.experimental.pallas import tpu as pltpu
from jax.experimental.pallas import tpu_sc as plsc

N_EMBD = 64
T = 200
B = 1024
NW = 32
LANES = 16
NBT = B // 128
NCH = T * NBT
CPW = NCH // NW
NTT = 9
TPAD = T + NTT


def _body(y0_hbm, y1_hbm, tab0_hbm, tab1_hbm, pos_hbm,
          z0_hbm, z1_hbm,
          ibuf, gbuf, obuf, pos_v,
          g0, g1, o0, o1, i0, i1, i2, i3):
    wid = lax.axis_index("s") * 2 + lax.axis_index("c")
    cbase = wid * CPW
    t_first = (wid * CPW) // NBT

    gsems = (g0, g1)
    osems = (o0, o1)
    isems = (i0, i1, i2, i3)

    pltpu.sync_copy(
        pos_hbm.at[pl.ds(t_first * N_EMBD, NTT * N_EMBD)], pos_v)

    iota = lax.iota(jnp.int32, LANES)

    for y_hbm, tab_hbm, z_hbm in (
        (y0_hbm, tab0_hbm, z0_hbm),
        (y1_hbm, tab1_hbm, z1_hbm),
    ):
        def coords(j):
            c = cbase + j
            return c // NBT, lax.rem(c, NBT)

        def fetch_idx(j, s):
            t, bt = coords(j)
            row = (t // 8) * 64 + bt * 8 + lax.rem(t, 8)
            pltpu.async_copy(y_hbm.at[pl.ds(row, 1)],
                             ibuf.at[pl.ds(s, 1)], isems[s])

        def start_gather(s, p):
            pltpu.make_async_copy(
                y_hbm.at[pl.ds(0, 1)], ibuf.at[pl.ds(0, 1)], isems[s]).wait()
            pltpu.async_copy(tab_hbm.at[ibuf.at[s]], gbuf.at[p], gsems[p])

        def consume(j, p):
            t, bt = coords(j)
            pltpu.make_async_copy(
                tab_hbm.at[ibuf.at[0]], gbuf.at[p], gsems[p]).wait()
            pb = (t - t_first) * N_EMBD

            @plsc.parallel_loop(0, N_EMBD, unroll=2)
            def _(u):
                eg = u // LANES
                d = lax.rem(u, LANES)
                ecol = eg * LANES + ((iota + d) & (LANES - 1))
                pvec = plsc.load_gather(pos_v, [pb + ecol])
                for bg in range(8):
                    rows = iota + (bg * LANES)
                    v = plsc.load_gather(gbuf.at[p], [rows, ecol])
                    plsc.store_scatter(obuf.at[p], [ecol, rows], v + pvec)

            tile0 = (t * 8) * NBT + bt
            for et in range(8):
                pltpu.async_copy(
                    obuf.at[p, pl.ds(et * 8, 8)],
                    z_hbm.at[pl.ds((tile0 + et * NBT) * 8, 8)], osems[p])

        def wait_out(p):
            for et in range(8):
                pltpu.make_async_copy(
                    obuf.at[p, pl.ds(0, 8)], z_hbm.at[pl.ds(0, 8)],
                    osems[p]).wait()

        for j in range(4):
            fetch_idx(j, j)
        for j in range(2):
            start_gather(j, j)

        @pl.loop(0, CPW - 2, step=4)
        def _(g):
            for k in range(4):
                j = g + k
                p = k % 2

                @pl.when(j >= 2)
                def _():
                    wait_out(p)

                consume(j, p)

                @pl.when(j + 4 < CPW)
                def _():
                    fetch_idx(j + 4, k)

                start_gather((k + 2) % 4, p)

        for p in range(2):
            wait_out(p)
            consume(CPW - 2 + p, p)
        for p in range(2):
            wait_out(p)


@jax.jit
def _run(y0, y1, table0, table1, pos_lin):
    mesh = plsc.VectorSubcoreMesh(core_axis_name="c", subcore_axis_name="s")
    zshape = jax.ShapeDtypeStruct((T * 8 * NBT * 8, 128), jnp.float32)
    f = pl.kernel(
        _body,
        out_type=[zshape, zshape],
        mesh=mesh,
        compiler_params=pltpu.CompilerParams(
            use_tc_tiling_on_sc=False, needs_layout_passes=False),
        scratch_types=[
            pltpu.VMEM((4, 128), jnp.int32),
            pltpu.VMEM((2, 128, N_EMBD), jnp.float32),
            pltpu.VMEM((2, N_EMBD, 128), jnp.float32),
            pltpu.VMEM((NTT * N_EMBD,), jnp.float32),
            pltpu.SemaphoreType.DMA,
            pltpu.SemaphoreType.DMA,
            pltpu.SemaphoreType.DMA,
            pltpu.SemaphoreType.DMA,
            pltpu.SemaphoreType.DMA,
            pltpu.SemaphoreType.DMA,
            pltpu.SemaphoreType.DMA,
            pltpu.SemaphoreType.DMA,
        ],
    )
    return f(y0, y1, table0, table1, pos_lin)


def kernel(idx0, idx1, table0, table1, pos_table):
    def as_tiles(idx):
        idx = idx.astype(jnp.int32)
        return (idx.T.reshape(T // 8, 8, NBT, 128)
                .transpose(0, 2, 1, 3).reshape(NCH, 128))

    pos_lin = jnp.pad(pos_table, ((0, NTT), (0, 0))).reshape(-1)

    z0, z1 = _run(as_tiles(idx0), as_tiles(idx1), table0, table1, pos_lin)

    def as_out(z):
        return (z.reshape(T, 8, NBT, 8, 128)
                .transpose(2, 4, 0, 1, 3).reshape(B, T, N_EMBD))

    return (as_out(z0), as_out(z1))


# speedup vs baseline: 1.5582x; 1.0376x over previous
import jax
import jax.numpy as jnp
from jax import lax
from jax.experimental import pallas as pl
from jax.experimental.pallas import tpu as pltpu
from jax.experimental.pallas import tpu_sc as plsc

N_EMBD = 64
T = 200
B = 1024
NW = 32
LANES = 16
NBT = B // 128
NCH = T * NBT
CPW = NCH // NW
NTT = 9
TPAD = T + NTT


def _body(y_hbm, tab_hbm, pos_hbm,
          z_hbm,
          ibuf, gbuf, obuf, pos_v,
          g0, g1, o0, o1, i0, i1, i2, i3):
    wid = lax.axis_index("s") * 2 + lax.axis_index("c")
    cbase = wid * CPW
    t_first = (wid * CPW) // NBT

    gsems = (g0, g1)
    osems = (o0, o1)
    isems = (i0, i1, i2, i3)

    pltpu.sync_copy(
        pos_hbm.at[pl.ds(t_first * N_EMBD, NTT * N_EMBD)], pos_v)

    iota = lax.iota(jnp.int32, LANES)

    if True:
        def coords(j):
            c = cbase + j
            return c // NBT, lax.rem(c, NBT)

        def fetch_idx(j, s):
            t, bt = coords(j)
            row = (t // 8) * 64 + bt * 8 + lax.rem(t, 8)
            pltpu.async_copy(y_hbm.at[pl.ds(row, 1)],
                             ibuf.at[pl.ds(s, 1)], isems[s])

        def start_gather(s, p):
            pltpu.make_async_copy(
                y_hbm.at[pl.ds(0, 1)], ibuf.at[pl.ds(0, 1)], isems[s]).wait()
            pltpu.async_copy(tab_hbm.at[ibuf.at[s]], gbuf.at[p], gsems[p])

        def consume(j, p):
            t, bt = coords(j)
            pltpu.make_async_copy(
                tab_hbm.at[ibuf.at[0]], gbuf.at[p], gsems[p]).wait()
            pb = (t - t_first) * N_EMBD

            @plsc.parallel_loop(0, N_EMBD, unroll=2)
            def _(u):
                eg = u // LANES
                d = lax.rem(u, LANES)
                ecol = eg * LANES + ((iota + d) & (LANES - 1))
                pvec = plsc.load_gather(pos_v, [pb + ecol])
                for bg in range(8):
                    rows = iota + (bg * LANES)
                    v = plsc.load_gather(gbuf.at[p], [rows, ecol])
                    plsc.store_scatter(obuf.at[p], [ecol, rows], v + pvec)

            tile0 = (t * 8) * NBT + bt
            for et in range(8):
                pltpu.async_copy(
                    obuf.at[p, pl.ds(et * 8, 8)],
                    z_hbm.at[pl.ds((tile0 + et * NBT) * 8, 8)], osems[p])

        def wait_out(p):
            for et in range(8):
                pltpu.make_async_copy(
                    obuf.at[p, pl.ds(0, 8)], z_hbm.at[pl.ds(0, 8)],
                    osems[p]).wait()

        for j in range(4):
            fetch_idx(j, j)
        for j in range(2):
            start_gather(j, j)

        @pl.loop(0, CPW - 2, step=4)
        def _(g):
            for k in range(4):
                j = g + k
                p = k % 2

                @pl.when(j >= 2)
                def _():
                    wait_out(p)

                consume(j, p)

                @pl.when(j + 4 < CPW)
                def _():
                    fetch_idx(j + 4, k)

                start_gather((k + 2) % 4, p)

        for p in range(2):
            wait_out(p)
            consume(CPW - 2 + p, p)
        for p in range(2):
            wait_out(p)


@jax.jit
def _run(y0, y1, table0, table1, pos_lin):
    mesh = plsc.VectorSubcoreMesh(core_axis_name="c", subcore_axis_name="s")
    zshape = jax.ShapeDtypeStruct((T * 8 * NBT * 8, 128), jnp.float32)
    f = pl.kernel(
        _body,
        out_type=[zshape],
        mesh=mesh,
        compiler_params=pltpu.CompilerParams(
            use_tc_tiling_on_sc=False, needs_layout_passes=False),
        scratch_types=[
            pltpu.VMEM((4, 128), jnp.int32),
            pltpu.VMEM((2, 128, N_EMBD), jnp.float32),
            pltpu.VMEM((2, N_EMBD, 128), jnp.float32),
            pltpu.VMEM((NTT * N_EMBD,), jnp.float32),
            pltpu.SemaphoreType.DMA,
            pltpu.SemaphoreType.DMA,
            pltpu.SemaphoreType.DMA,
            pltpu.SemaphoreType.DMA,
            pltpu.SemaphoreType.DMA,
            pltpu.SemaphoreType.DMA,
            pltpu.SemaphoreType.DMA,
            pltpu.SemaphoreType.DMA,
        ],
    )
    (z1,) = f(y1, table1, pos_lin)
    (z0,) = f(y0, table0, pos_lin)
    return z0, z1


def kernel(idx0, idx1, table0, table1, pos_table):
    def as_tiles(idx):
        idx = idx.astype(jnp.int32)
        return (idx.T.reshape(T // 8, 8, NBT, 128)
                .transpose(0, 2, 1, 3).reshape(NCH, 128))

    pos_lin = jnp.pad(pos_table, ((0, NTT), (0, 0))).reshape(-1)

    z0, z1 = _run(as_tiles(idx0), as_tiles(idx1), table0, table1, pos_lin)

    def as_out(z):
        return (z.reshape(T, 8, NBT, 8, 128)
                .transpose(2, 4, 0, 1, 3).reshape(B, T, N_EMBD))

    return (as_out(z0), as_out(z1))


# speedup vs baseline: 2.8551x; 1.8323x over previous
import jax
import jax.numpy as jnp
from jax import lax
from jax.experimental import pallas as pl
from jax.experimental.pallas import tpu as pltpu
from jax.experimental.pallas import tpu_sc as plsc

N_EMBD = 64
T = 200
B = 1024
NW = 32
LANES = 16
NBT = B // 128
NCH = T * NBT
CPW = NCH // NW
NTT = 9
TPAD = T + NTT


def _body(y_hbm, tab_hbm, pos_hbm,
          z_hbm,
          ibuf, pairb, hbuf, gbuf, obuf, pos_v,
          g0, g1, o0, o1, i0, i1, i2, i3):
    wid = lax.axis_index("s") * 2 + lax.axis_index("c")
    cbase = wid * CPW
    t_first = (wid * CPW) // NBT

    gsems = (g0, g1)
    osems = (o0, o1)
    isems = (i0, i1, i2, i3)

    pltpu.sync_copy(
        pos_hbm.at[pl.ds(t_first * N_EMBD, NTT * N_EMBD)], pos_v)

    iota = lax.iota(jnp.int32, LANES)

    if True:
        def coords(j):
            c = cbase + j
            return c // NBT, lax.rem(c, NBT)

        def fetch_idx(j, s):
            t, bt = coords(j)
            row = (t // 8) * 64 + bt * 8 + lax.rem(t, 8)
            pltpu.async_copy(y_hbm.at[pl.ds(row, 1)],
                             ibuf.at[pl.ds(s, 1)], isems[s])

        def start_gather(s, p):
            pltpu.make_async_copy(
                y_hbm.at[pl.ds(0, 1)], ibuf.at[pl.ds(0, 1)], isems[s]).wait()
            for r in range(8):
                sl = pl.ds(r * LANES, LANES)
                iv = ibuf[s, sl]
                pairb[s, sl] = lax.shift_right_logical(iv, 1)
                hbuf[s, sl] = lax.shift_left(iv & 1, 6)
            pltpu.async_copy(tab_hbm.at[pairb.at[s]], gbuf.at[p], gsems[p])

        def consume(j, s, p):
            t, bt = coords(j)
            pltpu.make_async_copy(
                tab_hbm.at[pairb.at[0]], gbuf.at[p], gsems[p]).wait()
            pb = (t - t_first) * N_EMBD

            @plsc.parallel_loop(0, N_EMBD, unroll=2)
            def _(u):
                eg = u // LANES
                d = lax.rem(u, LANES)
                ecol = eg * LANES + ((iota + d) & (LANES - 1))
                pvec = plsc.load_gather(pos_v, [pb + ecol])
                for bg in range(8):
                    rows = iota + (bg * LANES)
                    half = hbuf[s, pl.ds(bg * LANES, LANES)]
                    v = plsc.load_gather(gbuf.at[p], [rows, half + ecol])
                    plsc.store_scatter(obuf.at[p], [ecol, rows], v + pvec)

            tile0 = (t * 8) * NBT + bt
            for et in range(8):
                pltpu.async_copy(
                    obuf.at[p, pl.ds(et * 8, 8)],
                    z_hbm.at[pl.ds((tile0 + et * NBT) * 8, 8)], osems[p])

        def wait_out(p):
            for et in range(8):
                pltpu.make_async_copy(
                    obuf.at[p, pl.ds(0, 8)], z_hbm.at[pl.ds(0, 8)],
                    osems[p]).wait()

        for j in range(4):
            fetch_idx(j, j)
        for j in range(2):
            start_gather(j, j)

        @pl.loop(0, CPW - 2, step=4)
        def _(g):
            for k in range(4):
                j = g + k
                p = k % 2

                @pl.when(j >= 2)
                def _():
                    wait_out(p)

                consume(j, k, p)

                @pl.when(j + 4 < CPW)
                def _():
                    fetch_idx(j + 4, k)

                start_gather((k + 2) % 4, p)

        for p in range(2):
            wait_out(p)
            consume(CPW - 2 + p, (CPW - 2 + p) % 4, p)
        for p in range(2):
            wait_out(p)


CR = 256
V0 = 1000000
V1 = 100000


def _cbody(t0t_hbm, t1t_hbm, t0tail_hbm, t1tail_hbm, p0_hbm, p1_hbm,
           tib, cob, ci0, ci1, co0, co1):
    wid = lax.axis_index("s") * 2 + lax.axis_index("c")
    iota = lax.iota(jnp.int32, LANES)
    ihalf = lax.shift_right_logical(iota, 1)
    icol = lax.shift_left(iota & 1, 6)
    cisems = (ci0, ci1)
    cosems = (co0, co1)

    for tab_hbm, tail_hbm, p_hbm, V in ((t1t_hbm, t1tail_hbm, p1_hbm, V1),
                                        (t0t_hbm, t0tail_hbm, p0_hbm, V0)):
        full = V // CR
        base = full // NW
        extra = full - base * NW
        r_tail = full * CR
        l_tail = V - r_tail

        def dma_in(r0, q, w=CR):
            pltpu.async_copy(tab_hbm.at[:, pl.ds(r0, w)],
                             tib.at[q, :, pl.ds(0, w)], cisems[q])

        def wait_in(q, w=CR):
            pltpu.make_async_copy(tab_hbm.at[:, pl.ds(0, w)],
                                  tib.at[q, :, pl.ds(0, w)], cisems[q]).wait()

        def dma_out(pr0, q, rows=CR // 2):
            pltpu.async_copy(cob.at[q, pl.ds(0, rows)],
                             p_hbm.at[pl.ds(pr0, rows)], cosems[q])

        def wait_out(q, rows=CR // 2):
            pltpu.make_async_copy(cob.at[q, pl.ds(0, rows)],
                                  p_hbm.at[pl.ds(0, rows)], cosems[q]).wait()

        def pass_(q, steps):
            @plsc.parallel_loop(0, N_EMBD, unroll=2)
            def _(u):
                e0 = (u // LANES) * LANES
                d = lax.rem(u, LANES)
                evec = e0 + ((iota + d) & (LANES - 1))
                cvec = icol + evec
                for rt, nrs in steps:
                    rbase = iota + rt * 128
                    pbase = ihalf + rt * 64
                    for rs in range(nrs):
                        rvec = rbase + rs * LANES
                        v = plsc.load_gather(tib.at[q], [evec, rvec])
                        plsc.store_scatter(cob.at[q],
                                           [pbase + rs * 8, cvec], v)

        fsteps = ((0, 8), (1, 8))
        cb = wid * base

        dma_in(cb * CR, 0)
        dma_in((cb + 1) * CR, 1)

        @pl.loop(0, base, step=2)
        def _(i):
            for q in range(2):
                b = i + q
                wait_in(q)

                @pl.when(b >= 2)
                def _():
                    wait_out(q)

                pass_(q, fsteps)
                dma_out((cb + b) * (CR // 2), q)

                @pl.when(b + 2 < base)
                def _():
                    dma_in((cb + b + 2) * CR, q)

        for q in range(2):
            wait_out(q)

        @pl.when(wid < extra)
        def _():
            g = base * NW + wid
            dma_in(g * CR, 0)
            wait_in(0)
            pass_(0, fsteps)
            dma_out(g * (CR // 2), 0)
            wait_out(0)

        if l_tail:
            tr = l_tail // 2

            @pl.when(wid == extra)
            def _():
                pltpu.async_copy(tail_hbm, cob.at[0, pl.ds(0, tr)],
                                 cisems[0])
                pltpu.make_async_copy(tail_hbm, cob.at[0, pl.ds(0, tr)],
                                      cisems[0]).wait()
                dma_out(r_tail // 2, 0, tr)
                wait_out(0, tr)


@jax.jit
def _run(y0, y1, table0t, table1t, t0tail, t1tail, pos_lin):
    mesh = plsc.VectorSubcoreMesh(core_axis_name="c", subcore_axis_name="s")
    fc = pl.kernel(
        _cbody,
        out_type=[
            jax.ShapeDtypeStruct((V0 // 2, 128), jnp.float32),
            jax.ShapeDtypeStruct((V1 // 2, 128), jnp.float32),
        ],
        mesh=mesh,
        compiler_params=pltpu.CompilerParams(
            use_tc_tiling_on_sc=True, needs_layout_passes=False),
        scratch_types=[
            pltpu.VMEM((2, N_EMBD, CR), jnp.float32),
            pltpu.VMEM((2, CR // 2, 128), jnp.float32),
            pltpu.SemaphoreType.DMA,
            pltpu.SemaphoreType.DMA,
            pltpu.SemaphoreType.DMA,
            pltpu.SemaphoreType.DMA,
        ],
    )
    tab0p, tab1p = fc(table0t, table1t, t0tail, t1tail)
    zshape = jax.ShapeDtypeStruct((T * 8 * NBT * 8, 128), jnp.float32)
    f = pl.kernel(
        _body,
        out_type=[zshape],
        mesh=mesh,
        compiler_params=pltpu.CompilerParams(
            use_tc_tiling_on_sc=False, needs_layout_passes=False),
        scratch_types=[
            pltpu.VMEM((4, 128), jnp.int32),
            pltpu.VMEM((4, 128), jnp.int32),
            pltpu.VMEM((4, 128), jnp.int32),
            pltpu.VMEM((2, 128, 2 * N_EMBD), jnp.float32),
            pltpu.VMEM((2, N_EMBD, 128), jnp.float32),
            pltpu.VMEM((NTT * N_EMBD,), jnp.float32),
            pltpu.SemaphoreType.DMA,
            pltpu.SemaphoreType.DMA,
            pltpu.SemaphoreType.DMA,
            pltpu.SemaphoreType.DMA,
            pltpu.SemaphoreType.DMA,
            pltpu.SemaphoreType.DMA,
            pltpu.SemaphoreType.DMA,
            pltpu.SemaphoreType.DMA,
        ],
    )
    (z1,) = f(y1, tab1p, pos_lin)
    (z0,) = f(y0, tab0p, pos_lin)
    return z0, z1


def kernel(idx0, idx1, table0, table1, pos_table):
    def as_tiles(idx):
        idx = idx.astype(jnp.int32)
        return (idx.T.reshape(T // 8, 8, NBT, 128)
                .transpose(0, 2, 1, 3).reshape(NCH, 128))

    pos_lin = jnp.pad(pos_table, ((0, NTT), (0, 0))).reshape(-1)

    nf0 = (V0 // CR) * CR
    nf1 = (V1 // CR) * CR
    z0, z1 = _run(as_tiles(idx0), as_tiles(idx1), table0.T, table1.T,
                  table0[nf0:].reshape(-1, 128),
                  table1[nf1:].reshape(-1, 128), pos_lin)

    def as_out(z):
        return (z.reshape(T, 8, NBT, 8, 128)
                .transpose(2, 4, 0, 1, 3).reshape(B, T, N_EMBD))

    return (as_out(z0), as_out(z1))
